# baseline (speedup 1.0000x reference)
import jax
import jax.numpy as jnp
from jax.experimental import pallas as pl
from jax.experimental.pallas import tpu as pltpu

N = 10000
E = 160000
DT = 1.0

GRID = 5
NODE_BLOCK = N // GRID
EDGE_BLOCK = E // GRID


def _fused_kernel(xa_ref, xb_ref, w1_ref, b1_ref, w2_ref, b2_ref, wdn_ref, bdn_ref,
                  et_ref, wdet_ref, bdet_ref, ona_ref, onb_ref, oet_ref):
    for x_ref, o_ref in ((xa_ref, ona_ref), (xb_ref, onb_ref)):
        x = x_ref[...]
        h = jnp.dot(x, w1_ref[...], preferred_element_type=jnp.float32) + b1_ref[...]
        h = h * jax.nn.sigmoid(h)
        h = jnp.dot(h, w2_ref[...], preferred_element_type=jnp.float32) + b2_ref[...]
        h = h * jax.nn.sigmoid(h)
        o_ref[...] = jnp.dot(h, wdn_ref[...], preferred_element_type=jnp.float32) + bdn_ref[...]
    oet_ref[...] = (jnp.dot(wdet_ref[...], et_ref[...], preferred_element_type=jnp.float32)
                    + bdet_ref[...])


def kernel(nodes, edges, senders, receivers, globals_, W_enc_n, b_enc_n, W_enc_e, b_enc_e, W1, b1, W2, b2, Wd_n, bd_n, Wd_e, bd_e):
    d_feat = nodes.shape[1]
    latent = W1.shape[1]
    node_out = Wd_n.shape[1]
    d_edge = edges.shape[1]
    edge_out = Wd_e.shape[1]

    half = N // 2
    nodes_a, nodes_b = nodes[:half], nodes[half:]
    edges_t = edges.T
    wde_t = Wd_e.T
    bde_c = bd_e.reshape(-1, 1)

    whole = lambda *shape: pl.BlockSpec(shape, lambda i: (0,) * len(shape))

    out_nodes_a, out_nodes_b, out_edges_t = pl.pallas_call(
        _fused_kernel,
        grid=(GRID,),
        in_specs=[
            pl.BlockSpec((NODE_BLOCK // 2, d_feat), lambda i: (i, 0)),
            pl.BlockSpec((NODE_BLOCK // 2, d_feat), lambda i: (i, 0)),
            whole(d_feat, latent),
            whole(1, latent),
            whole(latent, latent),
            whole(1, latent),
            whole(latent, node_out),
            whole(1, node_out),
            pl.BlockSpec((d_edge, EDGE_BLOCK), lambda i: (0, i)),
            whole(edge_out, d_edge),
            whole(edge_out, 1),
        ],
        out_specs=[
            pl.BlockSpec((NODE_BLOCK // 2, node_out), lambda i: (i, 0)),
            pl.BlockSpec((NODE_BLOCK // 2, node_out), lambda i: (i, 0)),
            pl.BlockSpec((edge_out, EDGE_BLOCK), lambda i: (0, i)),
        ],
        out_shape=[
            jax.ShapeDtypeStruct((N // 2, node_out), jnp.float32),
            jax.ShapeDtypeStruct((N // 2, node_out), jnp.float32),
            jax.ShapeDtypeStruct((edge_out, E), jnp.float32),
        ],
        compiler_params=pltpu.CompilerParams(
            dimension_semantics=("arbitrary",),
        ),
    )(nodes_a, nodes_b, W1, b1.reshape(1, -1), W2, b2.reshape(1, -1), Wd_n, bd_n.reshape(1, -1),
      edges_t, wde_t, bde_c)

    out_nodes = jnp.concatenate([out_nodes_a, out_nodes_b], axis=0)

    out_edges = out_edges_t.T
    new_globals = globals_ + DT
    return out_nodes, out_edges, new_globals


# speedup vs baseline: 1.6735x; 1.6735x over previous
import jax
import jax.numpy as jnp
from jax.experimental import pallas as pl
from jax.experimental.pallas import tpu as pltpu

N = 10000
E = 160000
DT = 1.0

GRID = 5
NODE_BLOCK = N // GRID
EDGE_BLOCK = E // GRID


def _fused_kernel(x_ref, w1_ref, b1_ref, w2_ref, b2_ref, wdn_ref, bdn_ref,
                  et_ref, wdet_ref, bdet_ref, on_ref, oet_ref):
    x = x_ref[...]
    h = jnp.dot(x, w1_ref[...], preferred_element_type=jnp.float32) + b1_ref[...]
    h = h * jax.nn.sigmoid(h)
    h = jnp.dot(h, w2_ref[...], preferred_element_type=jnp.float32) + b2_ref[...]
    h = h * jax.nn.sigmoid(h)
    on_ref[...] = jnp.dot(h, wdn_ref[...], preferred_element_type=jnp.float32) + bdn_ref[...]
    oet_ref[...] = (jnp.dot(wdet_ref[...], et_ref[...], preferred_element_type=jnp.float32)
                    + bdet_ref[...])


def kernel(nodes, edges, senders, receivers, globals_, W_enc_n, b_enc_n, W_enc_e, b_enc_e, W1, b1, W2, b2, Wd_n, bd_n, Wd_e, bd_e):
    d_feat = nodes.shape[1]
    latent = W1.shape[1]
    node_out = Wd_n.shape[1]
    d_edge = edges.shape[1]
    edge_out = Wd_e.shape[1]

    edges_t = edges.T
    wde_t = Wd_e.T
    bde_c = bd_e.reshape(-1, 1)

    whole = lambda *shape: pl.BlockSpec(shape, lambda i: (0,) * len(shape))

    out_nodes, out_edges_t = pl.pallas_call(
        _fused_kernel,
        grid=(GRID,),
        in_specs=[
            pl.BlockSpec((NODE_BLOCK, d_feat), lambda i: (i, 0)),
            whole(d_feat, latent),
            whole(1, latent),
            whole(latent, latent),
            whole(1, latent),
            whole(latent, node_out),
            whole(1, node_out),
            pl.BlockSpec((d_edge, EDGE_BLOCK), lambda i: (0, i)),
            whole(edge_out, d_edge),
            whole(edge_out, 1),
        ],
        out_specs=[
            pl.BlockSpec((NODE_BLOCK, node_out), lambda i: (i, 0)),
            pl.BlockSpec((edge_out, EDGE_BLOCK), lambda i: (0, i)),
        ],
        out_shape=[
            jax.ShapeDtypeStruct((N, node_out), jnp.float32),
            jax.ShapeDtypeStruct((edge_out, E), jnp.float32),
        ],
        compiler_params=pltpu.CompilerParams(
            dimension_semantics=("arbitrary",),
        ),
    )(nodes, W1, b1.reshape(1, -1), W2, b2.reshape(1, -1), Wd_n, bd_n.reshape(1, -1),
      edges_t, wde_t, bde_c)

    out_edges = out_edges_t.T
    new_globals = globals_ + DT
    return out_nodes, out_edges, new_globals
